# baseline (speedup 1.0000x reference)
import functools

import jax
import jax.numpy as jnp
from jax import lax
from jax.experimental import pallas as pl
from jax.experimental.pallas import tpu as pltpu
from jax.experimental.pallas import tpu_sc as plsc

_B = 16
_P = 8732
_C = 21
_O = 5
_R = 69
_L = 128
_PP = _R * _L
_H = _PP // 2
_NV = _H // 16
_THRESHOLD = 0.5
_NEGPOS = 3
_V0 = 0.1
_V1 = 0.2
_NPIX = _B * (2500 + 625 + 169)


def _sc_match_kernel(pri_hbm, tw_hbm, conf_hbm, bti_hbm,
                     pri_v, tw_v, bto_v, bti_v, conf_s, btio_s,
                     shm, shi):
    c = lax.axis_index("c")
    s = lax.axis_index("s")
    img = c * 8 + s // 2
    half = s % 2
    base = half * _H

    pltpu.sync_copy(pri_hbm.at[pl.ds(half * 5 * _H, 5 * _H)], pri_v)
    pltpu.sync_copy(tw_hbm.at[pl.ds(img * 400, 400)], tw_v)

    lane = lax.iota(jnp.int32, 16)
    NEG = jnp.full((16,), -1.0, jnp.float32)
    BIG = jnp.full((16,), _PP, jnp.int32)

    def body(i, carry):
        m0, m1, m2, m3, m4, i0, i1, i2, i3, i4 = carry
        off = i * 16
        pxmin = pri_v[pl.ds(off, 16)]
        pymin = pri_v[pl.ds(_H + off, 16)]
        pxmax = pri_v[pl.ds(2 * _H + off, 16)]
        pymax = pri_v[pl.ds(3 * _H + off, 16)]
        area_p = pri_v[pl.ds(4 * _H + off, 16)]
        gidx = base + off + lane
        bto = NEG
        bti = jnp.zeros((16,), jnp.int32)
        ms = [m0, m1, m2, m3, m4]
        ia = [i0, i1, i2, i3, i4]
        for j in range(_O):
            tx0 = tw_v[pl.ds((j * 5 + 0) * 16, 16)]
            ty0 = tw_v[pl.ds((j * 5 + 1) * 16, 16)]
            tx1 = tw_v[pl.ds((j * 5 + 2) * 16, 16)]
            ty1 = tw_v[pl.ds((j * 5 + 3) * 16, 16)]
            ix = jnp.maximum(
                jnp.minimum(tx1, pxmax) - jnp.maximum(tx0, pxmin), 0.0)
            iy = jnp.maximum(
                jnp.minimum(ty1, pymax) - jnp.maximum(ty0, pymin), 0.0)
            inter = ix * iy
            area_t = (tx1 - tx0) * (ty1 - ty0)
            iou = inter / (area_t + area_p - inter)
            upd = iou > bto
            bti = jnp.where(upd, j, bti)
            bto = jnp.where(upd, iou, bto)
            updm = iou > ms[j]
            ia[j] = jnp.where(updm, gidx, ia[j])
            ms[j] = jnp.where(updm, iou, ms[j])
        bto_v[pl.ds(off, 16)] = bto
        bti_v[pl.ds(off, 16)] = bti
        return (ms[0], ms[1], ms[2], ms[3], ms[4],
                ia[0], ia[1], ia[2], ia[3], ia[4])

    init = (NEG, NEG, NEG, NEG, NEG, BIG, BIG, BIG, BIG, BIG)
    out = lax.fori_loop(0, _NV, body, init)
    ms = out[0:5]
    ia = out[5:10]

    mvec = jnp.full((16,), -1.0, jnp.float32)
    ivec = BIG
    for j in range(_O):
        mj = ms[j]
        ij = ia[j]
        for k in (1, 2, 4, 8):
            pm = jnp.take(mj, lane ^ k)
            pi = jnp.take(ij, lane ^ k)
            better = (pm > mj) | ((pm == mj) & (pi < ij))
            mj = jnp.where(better, pm, mj)
            ij = jnp.where(better, pi, ij)
        mvec = jnp.where(lane == j, mj, mvec)
        ivec = jnp.where(lane == j, ij, ivec)

    conf_s[pl.ds(0, 16)] = mvec
    btio_s[pl.ds(0, 16)] = ivec
    pltpu.sync_copy(conf_s.at[pl.ds(0, 16)], shm.at[pl.ds(s * 16, 16)])
    pltpu.sync_copy(btio_s.at[pl.ds(0, 16)], shi.at[pl.ds(s * 16, 16)])
    plsc.subcore_barrier()
    sib = jnp.where(half == 0, s + 1, s - 1)
    pltpu.sync_copy(shm.at[pl.ds(sib * 16, 16)], conf_s.at[pl.ds(0, 16)])
    pltpu.sync_copy(shi.at[pl.ds(sib * 16, 16)], btio_s.at[pl.ds(0, 16)])
    m_o = conf_s[pl.ds(0, 16)]
    i_o = btio_s[pl.ds(0, 16)]

    better = (m_o > mvec) | ((m_o == mvec) & (i_o < ivec))
    bp = jnp.where(better, i_o, ivec)

    bps = [jnp.take(bp, jnp.full((16,), j, jnp.int32)) for j in range(_O)]
    tl = [tw_v[pl.ds((j * 5 + 4) * 16, 16)] for j in range(_O)]

    def body2(i, carry):
        off = i * 16
        bto = bto_v[pl.ds(off, 16)]
        bti = bti_v[pl.ds(off, 16)]
        gidx = base + off + lane
        for j in range(_O):
            cond = gidx == bps[j]
            bti = jnp.where(cond, j, bti)
            bto = jnp.where(cond, 2.0, bto)
        lab = jnp.zeros((16,), jnp.float32)
        for j in range(_O):
            lab = jnp.where(bti == j, tl[j], lab)
        conf = jnp.where((bto < _THRESHOLD) | (gidx >= _P), 0.0, lab + 1.0)
        conf_s[pl.ds(off, 16)] = conf
        btio_s[pl.ds(off, 16)] = bti
        return carry

    lax.fori_loop(0, _NV, body2, 0)
    pltpu.sync_copy(conf_s, conf_hbm.at[pl.ds(img * _PP + base, _H)])
    pltpu.sync_copy(btio_s, bti_hbm.at[pl.ds(img * _PP + base, _H)])


def _sc_match(priors_sc, targets_wide):
    mesh = plsc.VectorSubcoreMesh(core_axis_name="c", subcore_axis_name="s")
    f = functools.partial(
        pl.kernel,
        mesh=mesh,
        out_type=[
            jax.ShapeDtypeStruct((_B * _PP,), jnp.float32),
            jax.ShapeDtypeStruct((_B * _PP,), jnp.int32),
        ],
        scratch_types=[
            pltpu.VMEM((5 * _H,), jnp.float32),
            pltpu.VMEM((400,), jnp.float32),
            pltpu.VMEM((_H,), jnp.float32),
            pltpu.VMEM((_H,), jnp.int32),
            pltpu.VMEM((_H,), jnp.float32),
            pltpu.VMEM((_H,), jnp.int32),
            pltpu.VMEM_SHARED((256,), jnp.float32),
            pltpu.VMEM_SHARED((256,), jnp.int32),
        ],
    )(_sc_match_kernel)
    return f(priors_sc, targets_wide)


def _fused_kernel(tg_ref,
                  ys1_ref, xs1_ref, m1_ref,
                  ys2_ref, xs2_ref, m2_ref,
                  ys3_ref, xs3_ref, m3_ref,
                  pri_ref, loc_ref, conf_ref, ct_ref, bti_ref,
                  ll_ref, lc_ref, lm_ref,
                  v_scr, np_scr, acc_ref):
    b = pl.program_id(0)

    @pl.when(b == 0)
    def _():
        acc_ref[0] = 0.0
        acc_ref[1] = 0.0
        acc_ref[2] = 0.0
        acc_ref[3] = 0.0

    mpart = 0.0
    for ys_ref, xs_ref, m_ref, S in ((ys1_ref, xs1_ref, m1_ref, 50),
                                     (ys2_ref, xs2_ref, m2_ref, 25),
                                     (ys3_ref, xs3_ref, m3_ref, 13)):
        S2 = S * S
        x = m_ref[0]
        ys = ys_ref[...]
        xs = xs_ref[...]
        label = jnp.zeros((1, S2), jnp.float32)
        for j in range(_O):
            tx0 = tg_ref[b, j, 0]
            ty0 = tg_ref[b, j, 1]
            tx1 = tg_ref[b, j, 2]
            ty1 = tg_ref[b, j, 3]
            tl = tg_ref[b, j, 4]
            xmin = jnp.maximum(jnp.floor(S * tx0), 0.0)
            ymin = jnp.maximum(jnp.floor(S * ty0), 0.0)
            xmax = jnp.minimum(jnp.ceil(S * tx1 + 1.0), float(S))
            ymax = jnp.minimum(jnp.ceil(S * ty1 + 1.0), float(S))
            cond = ((ys >= ymin) & (ys < ymax) & (xs >= xmin) & (xs < xmax))
            label = jnp.where(cond, tl + 1.0, label)
        m = jnp.max(x, axis=0, keepdims=True)
        s = jnp.sum(jnp.exp(x - m), axis=0, keepdims=True)
        lse = jnp.log(s) + m
        ci = lax.broadcasted_iota(jnp.int32, (_C, S2), 0).astype(jnp.float32)
        picked = jnp.sum(jnp.where(ci == label, x, 0.0), axis=0, keepdims=True)
        mpart = mpart + jnp.sum(lse - picked)
    acc_ref[0] += mpart

    cx = pri_ref[0]
    cy = pri_ref[1]
    w = pri_ref[2]
    h = pri_ref[3]
    conf_t = ct_ref[0]
    bti = bti_ref[0]
    posb = conf_t > 0.0
    posf = jnp.where(posb, 1.0, 0.0)

    mt0 = jnp.zeros((_R, _L), jnp.float32)
    mt1 = jnp.zeros((_R, _L), jnp.float32)
    mt2 = jnp.zeros((_R, _L), jnp.float32)
    mt3 = jnp.zeros((_R, _L), jnp.float32)
    for j in range(_O):
        sel = bti == j
        mt0 = jnp.where(sel, tg_ref[b, j, 0], mt0)
        mt1 = jnp.where(sel, tg_ref[b, j, 1], mt1)
        mt2 = jnp.where(sel, tg_ref[b, j, 2], mt2)
        mt3 = jnp.where(sel, tg_ref[b, j, 3], mt3)

    g0 = ((mt0 + mt2) * 0.5 - cx) / (_V0 * w)
    g1 = ((mt1 + mt3) * 0.5 - cy) / (_V0 * h)
    g2 = jnp.log(jnp.maximum(mt2 - mt0, 1e-10) / w) / _V1
    g3 = jnp.log(jnp.maximum(mt3 - mt1, 1e-10) / h) / _V1
    sl = 0.0
    for k, g in ((0, g0), (1, g1), (2, g2), (3, g3)):
        d = loc_ref[0, k] - g
        ad = jnp.abs(d)
        sl = sl + jnp.where(ad < 1.0, 0.5 * d * d, ad - 0.5)
    acc_ref[1] += jnp.sum(jnp.where(posb, sl, 0.0))
    npsum = jnp.sum(posf)
    acc_ref[2] += npsum

    pidx = (lax.broadcasted_iota(jnp.int32, (_R, _L), 0) * _L
            + lax.broadcasted_iota(jnp.int32, (_R, _L), 1))
    valid = pidx < _P
    x = conf_ref[0]
    m = jnp.max(x, axis=0)
    s = jnp.sum(jnp.exp(x - m[None]), axis=0)
    lse = jnp.log(s) + m
    ci = lax.broadcasted_iota(jnp.int32, (_C, _R, _L), 0).astype(jnp.float32)
    picked = jnp.sum(jnp.where(ci == conf_t[None], x, 0.0), axis=0)
    ce = jnp.where(valid, lse - picked, 0.0)
    acc_ref[3] += jnp.sum(jnp.where(posb, ce, 0.0))
    v_scr[b] = jnp.where(posb, 0.0, ce)
    np_scr[b] = jnp.zeros((_L,), jnp.float32) + npsum

    @pl.when(b == _B - 1)
    def _():
        v = v_scr[...]
        bits = lax.bitcast_convert_type(v, jnp.int32)
        pidx3 = (lax.broadcasted_iota(jnp.int32, (_B, _R, _L), 1) * _L
                 + lax.broadcasted_iota(jnp.int32, (_B, _R, _L), 2))
        num_pos = np_scr[...][:, 0:1].reshape(_B, 1, 1)
        kk = jnp.minimum(_NEGPOS * num_pos, float(_P - 1))

        def srch(i, carry):
            lo, hi = carry
            mid = lo + (hi - lo + 1) // 2
            cnt = jnp.sum(jnp.where(bits >= mid, 1.0, 0.0), axis=(1, 2),
                          keepdims=True)
            ok = cnt >= kk
            return jnp.where(ok, mid, lo), jnp.where(ok, hi, mid - 1)

        lo0 = jnp.zeros((_B, 1, 1), jnp.int32)
        hi0 = jnp.full((_B, 1, 1), 0x7f800000, jnp.int32)
        t, _u1 = lax.fori_loop(0, 31, srch, (lo0, hi0))

        gt = bits > t
        eq = bits == t
        need = kk - jnp.sum(jnp.where(gt, 1.0, 0.0), axis=(1, 2),
                            keepdims=True)

        def srch2(i, carry):
            lo, hi = carry
            mid = (lo + hi) // 2
            cnt = jnp.sum(jnp.where(eq & (pidx3 <= mid), 1.0, 0.0),
                          axis=(1, 2), keepdims=True)
            ok = cnt >= need
            return jnp.where(ok, lo, mid + 1), jnp.where(ok, mid, hi)

        lo0 = jnp.zeros((_B, 1, 1), jnp.int32)
        hi0 = jnp.full((_B, 1, 1), _PP - 1, jnp.int32)
        idx_t, _u2 = lax.fori_loop(0, 14, srch2, (lo0, hi0))

        neg = gt | (eq & (pidx3 <= idx_t))
        negsum = jnp.sum(jnp.where(neg, v, 0.0))
        n = acc_ref[2]
        ll_ref[0, 0] = acc_ref[1] / n * 2.0
        lc_ref[0, 0] = (acc_ref[3] + negsum) / n * 2.0
        lm_ref[0, 0] = acc_ref[0] / float(_NPIX)


def _coords(S):
    p = jnp.arange(S * S, dtype=jnp.int32)
    return ((p // S).astype(jnp.float32).reshape(1, S * S),
            (p % S).astype(jnp.float32).reshape(1, S * S))


def kernel(loc_data, conf_data, priors, mask1, mask2, mask3, targets):
    pt = jnp.transpose(priors, (1, 0))
    cxr, cyr, wr, hr = pt[0], pt[1], pt[2], pt[3]
    rows = jnp.stack([cxr - wr * 0.5, cyr - hr * 0.5,
                      cxr + wr * 0.5, cyr + hr * 0.5, wr * hr], 0)
    padrow = jnp.tile(jnp.array([[2.5], [2.5], [3.0], [3.0], [0.25]],
                                jnp.float32), (1, _PP - _P))
    rows = jnp.concatenate([rows, padrow], axis=1)
    priors_sc = rows.reshape(5, 2, _H).transpose(1, 0, 2).reshape(-1)
    targets_wide = jnp.broadcast_to(
        targets.reshape(_B, 25)[:, :, None], (_B, 25, 16)).reshape(-1)

    conf_sc, bti_sc = _sc_match(priors_sc, targets_wide)
    ct = conf_sc.reshape(_B, _R, _L)
    bt = bti_sc.reshape(_B, _R, _L)

    conf_t = jnp.transpose(conf_data, (0, 2, 1))
    conf_t = jnp.pad(conf_t, ((0, 0), (0, 0), (0, _PP - _P)))
    conf_t = conf_t.reshape(_B, _C, _R, _L)
    loc_t = jnp.transpose(loc_data, (0, 2, 1))
    loc_t = jnp.pad(loc_t, ((0, 0), (0, 0), (0, _PP - _P)))
    loc_t = loc_t.reshape(_B, 4, _R, _L)
    pad_vals = jnp.tile(jnp.array([[3.0], [3.0], [1.0], [1.0]], jnp.float32),
                        (1, _PP - _P))
    pri_t = jnp.concatenate([pt, pad_vals], axis=1).reshape(4, _R, _L)
    m1 = mask1.reshape(_B, _C, 2500)
    m2 = mask2.reshape(_B, _C, 625)
    m3 = mask3.reshape(_B, _C, 169)
    ys1, xs1 = _coords(50)
    ys2, xs2 = _coords(25)
    ys3, xs3 = _coords(13)

    smem = pl.BlockSpec(memory_space=pltpu.SMEM)

    def cspec(shape):
        return pl.BlockSpec(shape, lambda b: tuple(0 for _ in shape))

    ll, lc, lm = pl.pallas_call(
        _fused_kernel,
        grid=(_B,),
        in_specs=[
            smem,
            cspec((1, 2500)), cspec((1, 2500)),
            pl.BlockSpec((1, _C, 2500), lambda b: (b, 0, 0)),
            cspec((1, 625)), cspec((1, 625)),
            pl.BlockSpec((1, _C, 625), lambda b: (b, 0, 0)),
            cspec((1, 169)), cspec((1, 169)),
            pl.BlockSpec((1, _C, 169), lambda b: (b, 0, 0)),
            cspec((4, _R, _L)),
            pl.BlockSpec((1, 4, _R, _L), lambda b: (b, 0, 0, 0)),
            pl.BlockSpec((1, _C, _R, _L), lambda b: (b, 0, 0, 0)),
            pl.BlockSpec((1, _R, _L), lambda b: (b, 0, 0)),
            pl.BlockSpec((1, _R, _L), lambda b: (b, 0, 0)),
        ],
        out_specs=[
            pl.BlockSpec(memory_space=pltpu.SMEM),
            pl.BlockSpec(memory_space=pltpu.SMEM),
            pl.BlockSpec(memory_space=pltpu.SMEM),
        ],
        out_shape=[
            jax.ShapeDtypeStruct((1, 1), jnp.float32),
            jax.ShapeDtypeStruct((1, 1), jnp.float32),
            jax.ShapeDtypeStruct((1, 1), jnp.float32),
        ],
        scratch_shapes=[
            pltpu.VMEM((_B, _R, _L), jnp.float32),
            pltpu.VMEM((_B, _L), jnp.float32),
            pltpu.SMEM((4,), jnp.float32),
        ],
    )(targets, ys1, xs1, m1, ys2, xs2, m2, ys3, xs3, m3,
      pri_t, loc_t, conf_t, ct, bt)

    return ll[0, 0], lc[0, 0], lm[0, 0]
